# Initial kernel scaffold; baseline (speedup 1.0000x reference)
import jax
import jax.numpy as jnp
from jax.experimental import pallas as pl


def kernel(z, y, protos, init_mask):
    raise NotImplementedError("write your pallas kernel here")


# speedup vs baseline: 1.0351x; 1.0351x over previous
import jax
import jax.numpy as jnp
from jax.experimental import pallas as pl

_MOM = 0.99
_BN = 512
_BC = 1024


def _segsum_body(y_ref, z_ref, sums_ref, counts_ref):
    i = pl.program_id(0)

    @pl.when(i == 0)
    def _init():
        sums_ref[...] = jnp.zeros_like(sums_ref)
        counts_ref[...] = jnp.zeros_like(counts_ref)

    y = y_ref[0]
    z = z_ref[...]
    n_cls = sums_ref.shape[0]
    classes = jax.lax.broadcasted_iota(jnp.int32, (n_cls, 1), 0)
    onehot_t = (classes == y).astype(jnp.float32)
    sums_ref[...] += jnp.dot(onehot_t, z, preferred_element_type=jnp.float32)
    counts_ref[...] += jnp.sum(onehot_t, axis=1, keepdims=True)


def _finish_body(sums_ref, counts_ref, protos_ref, mask_ref, out_ref):
    sums = sums_ref[...]
    counts = counts_ref[...]
    protos = protos_ref[...]
    mask = mask_ref[...]
    mean = sums / jnp.maximum(counts, 1.0)
    nrm = jnp.sqrt(jnp.sum(mean * mean, axis=1, keepdims=True))
    zc = mean / jnp.maximum(nrm, 1e-12)
    ema = _MOM * protos + (1.0 - _MOM) * zc
    enrm = jnp.sqrt(jnp.sum(ema * ema, axis=1, keepdims=True))
    ema = ema / jnp.maximum(enrm, 1e-12)
    new = jnp.where(mask > 0.0, ema, zc)
    out_ref[...] = jnp.where(counts > 0.0, new, protos)


def kernel(z, y, protos, init_mask):
    n, d = z.shape
    c = protos.shape[0]
    g = n // _BN
    y3 = y.reshape(g, 1, _BN).astype(jnp.int32)

    sums, counts = pl.pallas_call(
        _segsum_body,
        grid=(g,),
        in_specs=[
            pl.BlockSpec((1, 1, _BN), lambda i: (i, 0, 0)),
            pl.BlockSpec((_BN, d), lambda i: (i, 0)),
        ],
        out_specs=[
            pl.BlockSpec((c, d), lambda i: (0, 0)),
            pl.BlockSpec((c, 1), lambda i: (0, 0)),
        ],
        out_shape=[
            jax.ShapeDtypeStruct((c, d), jnp.float32),
            jax.ShapeDtypeStruct((c, 1), jnp.float32),
        ],
    )(y3, z)

    maskf = init_mask.astype(jnp.float32).reshape(c, 1)
    out = pl.pallas_call(
        _finish_body,
        grid=(c // _BC,),
        in_specs=[
            pl.BlockSpec((_BC, d), lambda i: (i, 0)),
            pl.BlockSpec((_BC, 1), lambda i: (i, 0)),
            pl.BlockSpec((_BC, d), lambda i: (i, 0)),
            pl.BlockSpec((_BC, 1), lambda i: (i, 0)),
        ],
        out_specs=pl.BlockSpec((_BC, d), lambda i: (i, 0)),
        out_shape=jax.ShapeDtypeStruct((c, d), jnp.float32),
    )(sums, counts, protos, maskf)
    return out


# speedup vs baseline: 4.1371x; 3.9968x over previous
import jax
import jax.numpy as jnp
from jax import lax
from jax.experimental import pallas as pl
from jax.experimental.pallas import tpu as pltpu
from jax.experimental.pallas import tpu_sc as plsc

_MOM = 0.99
_NC = 2
_NS = 16
_CH = 128
_BC = 1024


def _sc_segsum(z, y2, zeros_d):
    n, d = z.shape
    c = zeros_d.shape[0]
    hd = d // _NC
    rps = n // _NS
    nch = rps // _CH
    cps = c // _NS

    def body(z_hbm, y_hbm, zd_hbm, sums_out, acc, ybuf, zbuf):
        h = lax.axis_index("c")
        s = lax.axis_index("s")
        col0 = pl.multiple_of(h * hd, hd)
        crow0 = pl.multiple_of(s * cps, cps)

        pltpu.sync_copy(zd_hbm.at[pl.ds(crow0, cps)], acc.at[pl.ds(crow0, cps)])
        pltpu.sync_copy(y_hbm.at[pl.ds(pl.multiple_of(s * nch, nch), nch)], ybuf)
        plsc.subcore_barrier()

        row0 = pl.multiple_of(s * rps, rps)

        def chunk_step(j, carry):
            off = pl.multiple_of(row0 + j * _CH, _CH)
            pltpu.sync_copy(z_hbm.at[pl.ds(off, _CH), pl.ds(col0, hd)], zbuf)
            pltpu.sync_copy(zbuf, acc.at[ybuf.at[j]], add=True)
            return carry

        lax.fori_loop(0, nch, chunk_step, 0)
        plsc.subcore_barrier()
        pltpu.sync_copy(acc.at[pl.ds(crow0, cps)],
                        sums_out.at[pl.ds(crow0, cps), pl.ds(col0, hd)])

    mesh = plsc.VectorSubcoreMesh(core_axis_name="c", subcore_axis_name="s")
    fn = pl.kernel(
        body,
        mesh=mesh,
        out_type=jax.ShapeDtypeStruct((c, d), jnp.float32),
        scratch_types=[
            pltpu.VMEM_SHARED((c, hd), jnp.float32),
            pltpu.VMEM((nch, _CH), jnp.int32),
            pltpu.VMEM((_CH, hd), jnp.float32),
        ],
    )
    return fn(z, y2, zeros_d)


def _finish_body(sums_ref, protos_ref, mask_ref, out_ref):
    sums = sums_ref[...]
    protos = protos_ref[...]
    mask = mask_ref[...]
    nrm = jnp.sqrt(jnp.sum(sums * sums, axis=1, keepdims=True))
    zc = sums / jnp.maximum(nrm, 1e-12)
    ema = _MOM * protos + (1.0 - _MOM) * zc
    enrm = jnp.sqrt(jnp.sum(ema * ema, axis=1, keepdims=True))
    ema = ema / jnp.maximum(enrm, 1e-12)
    new = jnp.where(mask > 0.0, ema, zc)
    out_ref[...] = jnp.where(nrm > 0.0, new, protos)


def kernel(z, y, protos, init_mask):
    n, d = z.shape
    c = protos.shape[0]
    y2 = y.reshape(n // _CH, _CH).astype(jnp.int32)
    zeros_d = jnp.zeros((c, d // _NC), jnp.float32)

    sums = _sc_segsum(z, y2, zeros_d)

    maskf = init_mask.astype(jnp.float32).reshape(c, 1)
    out = pl.pallas_call(
        _finish_body,
        grid=(c // _BC,),
        in_specs=[
            pl.BlockSpec((_BC, d), lambda i: (i, 0)),
            pl.BlockSpec((_BC, d), lambda i: (i, 0)),
            pl.BlockSpec((_BC, 1), lambda i: (i, 0)),
        ],
        out_specs=pl.BlockSpec((_BC, d), lambda i: (i, 0)),
        out_shape=jax.ShapeDtypeStruct((c, d), jnp.float32),
    )(sums, protos, maskf)
    return out


# speedup vs baseline: 5.4941x; 1.3280x over previous
import jax
import jax.numpy as jnp
from jax import lax
from jax.experimental import pallas as pl
from jax.experimental.pallas import tpu as pltpu
from jax.experimental.pallas import tpu_sc as plsc

_MOM = 0.99
_NC = 2
_NS = 16
_CH = 128
_BC = 1024


def _sc_segsum(z, y2, zeros_d):
    n, d = z.shape
    c = zeros_d.shape[0]
    hd = d // _NC
    rps = n // _NS
    nch = rps // _CH
    cps = c // _NS

    nbuf = 2
    nout = nch // nbuf

    def body(z_hbm, y_hbm, zd_hbm, sums_out, acc, ybuf, *rest):
        zbufs = rest[:nbuf]
        gsems = rest[nbuf:]
        h = lax.axis_index("c")
        s = lax.axis_index("s")
        col0 = pl.multiple_of(h * hd, hd)
        crow0 = pl.multiple_of(s * cps, cps)
        row0 = pl.multiple_of(s * rps, rps)

        def gather(chunk, b):
            off = pl.multiple_of(row0 + chunk * _CH, _CH)
            pltpu.make_async_copy(
                z_hbm.at[pl.ds(off, _CH), pl.ds(col0, hd)],
                zbufs[b], gsems[b]).start()

        def gather_wait(b):
            pltpu.make_async_copy(
                z_hbm.at[pl.ds(row0, _CH), pl.ds(col0, hd)],
                zbufs[b], gsems[b]).wait()

        for b in range(nbuf):
            gather(b, b)
        pltpu.sync_copy(zd_hbm.at[pl.ds(crow0, cps)], acc.at[pl.ds(crow0, cps)])
        pltpu.sync_copy(y_hbm.at[pl.ds(pl.multiple_of(s * nch, nch), nch)], ybuf)
        plsc.subcore_barrier()

        def outer_step(t, carry):
            for b in range(nbuf):
                c_ = t * nbuf + b
                gather_wait(b)
                pltpu.sync_copy(zbufs[b], acc.at[ybuf.at[c_]], add=True)

                @pl.when(t < nout - 1)
                def _():
                    gather(c_ + nbuf, b)

            return carry

        lax.fori_loop(0, nout, outer_step, 0)
        plsc.subcore_barrier()
        pltpu.sync_copy(acc.at[pl.ds(crow0, cps)],
                        sums_out.at[pl.ds(crow0, cps), pl.ds(col0, hd)])

    mesh = plsc.VectorSubcoreMesh(core_axis_name="c", subcore_axis_name="s")
    fn = pl.kernel(
        body,
        mesh=mesh,
        out_type=jax.ShapeDtypeStruct((c, d), jnp.float32),
        scratch_types=[
            pltpu.VMEM_SHARED((c, hd), jnp.float32),
            pltpu.VMEM((nch, _CH), jnp.int32),
        ] + [pltpu.VMEM((_CH, hd), jnp.float32)] * nbuf
          + [pltpu.SemaphoreType.DMA] * nbuf,
    )
    return fn(z, y2, zeros_d)


def _finish_body(sums_ref, protos_ref, mask_ref, out_ref):
    sums = sums_ref[...]
    protos = protos_ref[...]
    mask = mask_ref[...]
    nrm = jnp.sqrt(jnp.sum(sums * sums, axis=1, keepdims=True))
    zc = sums / jnp.maximum(nrm, 1e-12)
    ema = _MOM * protos + (1.0 - _MOM) * zc
    enrm = jnp.sqrt(jnp.sum(ema * ema, axis=1, keepdims=True))
    ema = ema / jnp.maximum(enrm, 1e-12)
    new = jnp.where(mask > 0.0, ema, zc)
    out_ref[...] = jnp.where(nrm > 0.0, new, protos)


def kernel(z, y, protos, init_mask):
    n, d = z.shape
    c = protos.shape[0]
    y2 = y.reshape(n // _CH, _CH).astype(jnp.int32)
    zeros_d = jnp.zeros((c, d // _NC), jnp.float32)

    sums = _sc_segsum(z, y2, zeros_d)

    maskf = init_mask.astype(jnp.float32).reshape(c, 1)
    out = pl.pallas_call(
        _finish_body,
        grid=(c // _BC,),
        in_specs=[
            pl.BlockSpec((_BC, d), lambda i: (i, 0)),
            pl.BlockSpec((_BC, d), lambda i: (i, 0)),
            pl.BlockSpec((_BC, 1), lambda i: (i, 0)),
        ],
        out_specs=pl.BlockSpec((_BC, d), lambda i: (i, 0)),
        out_shape=jax.ShapeDtypeStruct((c, d), jnp.float32),
    )(sums, protos, maskf)
    return out
